# baseline (speedup 1.0000x reference)
import jax
import jax.numpy as jnp
from jax.experimental import pallas as pl

N_NODES = 100000
N_EDGES = 6400000

_EDGE_BLK = 2048
_NODE_BLK = 4000


def _edge_body(g_ref, ea_ref, w1ae_ref, w1b_ref, b1b_ref, msg_ref):
    g = g_ref[...]
    ea = ea_ref[...]
    e = jax.lax.dot_general(ea, w1ae_ref[...], (((1,), (0,)), ((), ())),
                            preferred_element_type=jnp.float32)
    h = jnp.maximum(g + e, 0.0)
    m = jax.lax.dot_general(h, w1b_ref[...], (((1,), (0,)), ((), ())),
                            preferred_element_type=jnp.float32)
    msg_ref[...] = jnp.maximum(m + b1b_ref[...], 0.0)


def _edge_mlp(g, ea, w1ae, w1b, b1b):
    nblk = N_EDGES // _EDGE_BLK
    return pl.pallas_call(
        _edge_body,
        grid=(nblk,),
        in_specs=[
            pl.BlockSpec((_EDGE_BLK, 16), lambda i: (i, 0)),
            pl.BlockSpec((_EDGE_BLK, 2), lambda i: (i, 0)),
            pl.BlockSpec((2, 16), lambda i: (0, 0)),
            pl.BlockSpec((16, 32), lambda i: (0, 0)),
            pl.BlockSpec((1, 32), lambda i: (0, 0)),
        ],
        out_specs=pl.BlockSpec((_EDGE_BLK, 32), lambda i: (i, 0)),
        out_shape=jax.ShapeDtypeStruct((N_EDGES, 32), jnp.float32),
    )(g, ea, w1ae, w1b, b1b)


def _node_body(x_ref, agg_ref, w2ax_ref, w2aa_ref, b2a_ref, w2b_ref, b2b_ref,
               w1a_ref, b1a_ref, comb_ref, np_ref):
    x = x_ref[...]
    agg = agg_ref[...]
    h = jax.lax.dot_general(x, w2ax_ref[...], (((1,), (0,)), ((), ())),
                            preferred_element_type=jnp.float32)
    h = h + jax.lax.dot_general(agg, w2aa_ref[...], (((1,), (0,)), ((), ())),
                                preferred_element_type=jnp.float32)
    h = jnp.maximum(h + b2a_ref[...], 0.0)
    comb = jax.lax.dot_general(h, w2b_ref[...], (((1,), (0,)), ((), ())),
                               preferred_element_type=jnp.float32)
    comb = jnp.maximum(comb + b2b_ref[...], 0.0)
    comb_ref[...] = comb
    x_next = jnp.concatenate([x[:, :11], comb], axis=1)
    npre = jax.lax.dot_general(x_next, w1a_ref[...], (((1,), (0,)), ((), ())),
                               preferred_element_type=jnp.float32)
    np_ref[...] = npre + b1a_ref[...]


def _node_mlp(x, agg, w2ax, w2aa, b2a, w2b, b2b, w1a, b1a):
    nblk = N_NODES // _NODE_BLK
    return pl.pallas_call(
        _node_body,
        grid=(nblk,),
        in_specs=[
            pl.BlockSpec((_NODE_BLK, 12), lambda i: (i, 0)),
            pl.BlockSpec((_NODE_BLK, 32), lambda i: (i, 0)),
            pl.BlockSpec((12, 16), lambda i: (0, 0)),
            pl.BlockSpec((32, 16), lambda i: (0, 0)),
            pl.BlockSpec((1, 16), lambda i: (0, 0)),
            pl.BlockSpec((16, 1), lambda i: (0, 0)),
            pl.BlockSpec((1, 1), lambda i: (0, 0)),
            pl.BlockSpec((12, 16), lambda i: (0, 0)),
            pl.BlockSpec((1, 16), lambda i: (0, 0)),
        ],
        out_specs=[
            pl.BlockSpec((_NODE_BLK, 1), lambda i: (i, 0)),
            pl.BlockSpec((_NODE_BLK, 16), lambda i: (i, 0)),
        ],
        out_shape=[
            jax.ShapeDtypeStruct((N_NODES, 1), jnp.float32),
            jax.ShapeDtypeStruct((N_NODES, 16), jnp.float32),
        ],
    )(x, agg, w2ax, w2aa, b2a, w2b, b2b, w1a, b1a)


def _pre_body(x_ref, w1a_ref, b1a_ref, np_ref):
    npre = jax.lax.dot_general(x_ref[...], w1a_ref[...], (((1,), (0,)), ((), ())),
                               preferred_element_type=jnp.float32)
    np_ref[...] = npre + b1a_ref[...]


def _node_pre(x, w1a, b1a):
    nblk = N_NODES // _NODE_BLK
    return pl.pallas_call(
        _pre_body,
        grid=(nblk,),
        in_specs=[
            pl.BlockSpec((_NODE_BLK, 12), lambda i: (i, 0)),
            pl.BlockSpec((12, 16), lambda i: (0, 0)),
            pl.BlockSpec((1, 16), lambda i: (0, 0)),
        ],
        out_specs=pl.BlockSpec((_NODE_BLK, 16), lambda i: (i, 0)),
        out_shape=jax.ShapeDtypeStruct((N_NODES, 16), jnp.float32),
    )(x, w1a, b1a)


def kernel(x, edge_index, edge_attr, W1a, b1a, W1b, b1b, W2a, b2a, W2b, b2b):
    src = edge_index[0].astype(jnp.int32)
    dst = edge_index[1].astype(jnp.int32)
    ea = edge_attr

    w1a_x = W1a[:12]
    w1a_e = W1a[12:14]
    b1a_r = b1a.reshape(1, 16)
    b1b_r = b1b.reshape(1, 32)
    w2a_x = W2a[:12]
    w2a_a = W2a[12:44]
    b2a_r = b2a.reshape(1, 16)
    b2b_r = b2b.reshape(1, 1)

    xc = x[:, :11]
    npre = _node_pre(x, w1a_x, b1a_r)
    x_cur = x
    for _ in range(3):
        g = jnp.take(npre, src, axis=0)
        msg = _edge_mlp(g, ea, w1a_e, W1b, b1b_r)
        agg = jax.ops.segment_max(msg, dst, num_segments=N_NODES)
        agg = jnp.maximum(agg, 0.0)
        comb, npre = _node_mlp(x_cur, agg, w2a_x, w2a_a, b2a_r, W2b,
                               b2b_r, w1a_x, b1a_r)
        x_cur = jnp.concatenate([xc, comb], axis=1)
    return x_cur


# speedup vs baseline: 2.0569x; 2.0569x over previous
import functools

import jax
import jax.numpy as jnp
from jax import lax
from jax.experimental import pallas as pl
from jax.experimental.pallas import tpu as pltpu
from jax.experimental.pallas import tpu_sc as plsc

N_NODES = 100000
N_EDGES = 6400000

_NW = 32
_GW = 400
_EPW = N_EDGES // _NW


def _sc_mesh():
    return plsc.VectorSubcoreMesh(core_axis_name="c", subcore_axis_name="s")


def _gather_body(table_hbm, src_hbm, out_hbm, src_v, bidx_v, rows_v,
                 h_v, sem):
    s = lax.axis_index("s")
    wid = s * 2 + lax.axis_index("c")
    base = wid * _EPW

    def step(j, _):
        off = base + j * _GW
        pltpu.sync_copy(src_hbm.at[pl.ds(off, _GW)], src_v)

        def mk_bidx(g, _c):
            v = src_v[pl.ds(g * 16, 16)]
            bidx_v[pl.ds(g * 16, 16)] = lax.shift_right_logical(v, 3)
            return 0

        lax.fori_loop(0, _GW // 16, mk_bidx, 0)
        pltpu.async_copy(table_hbm.at[bidx_v], rows_v, sem).wait()

        def extract(g, _c):
            v = src_v[pl.ds(g * 16, 16)]
            sub = lax.shift_left(lax.bitwise_and(v, 7), 4)
            for k in range(16):
                e = g * 16 + k
                h_v[e, :] = rows_v[e, pl.ds(sub[k], 16)]
            return 0

        lax.fori_loop(0, _GW // 16, extract, 0)
        pltpu.sync_copy(h_v, out_hbm.at[pl.ds(off, _GW)])
        return 0

    lax.fori_loop(0, _EPW // _GW, step, 0)


def _sc_gather(table_wide, src):
    fn = pl.kernel(
        _gather_body,
        out_type=jax.ShapeDtypeStruct((N_EDGES, 16), jnp.float32),
        mesh=_sc_mesh(),
        scratch_types=[
            pltpu.VMEM((_GW,), jnp.int32),
            pltpu.VMEM((_GW,), jnp.int32),
            pltpu.VMEM((_GW, 128), jnp.float32),
            pltpu.VMEM((_GW, 16), jnp.float32),
            pltpu.SemaphoreType.DMA,
        ],
    )
    return fn(table_wide, src)

_EDGE_BLK = 2048
_NODE_BLK = 4000


def _edge_body(g_ref, ea_ref, w1ae_ref, w1b_ref, b1b_ref, msg_ref):
    g = g_ref[...]
    ea = ea_ref[...]
    e = jax.lax.dot_general(ea, w1ae_ref[...], (((1,), (0,)), ((), ())),
                            preferred_element_type=jnp.float32)
    h = jnp.maximum(g + e, 0.0)
    m = jax.lax.dot_general(h, w1b_ref[...], (((1,), (0,)), ((), ())),
                            preferred_element_type=jnp.float32)
    msg_ref[...] = jnp.maximum(m + b1b_ref[...], 0.0)


def _edge_mlp(g, ea, w1ae, w1b, b1b):
    nblk = N_EDGES // _EDGE_BLK
    return pl.pallas_call(
        _edge_body,
        grid=(nblk,),
        in_specs=[
            pl.BlockSpec((_EDGE_BLK, 16), lambda i: (i, 0)),
            pl.BlockSpec((_EDGE_BLK, 2), lambda i: (i, 0)),
            pl.BlockSpec((2, 16), lambda i: (0, 0)),
            pl.BlockSpec((16, 32), lambda i: (0, 0)),
            pl.BlockSpec((1, 32), lambda i: (0, 0)),
        ],
        out_specs=pl.BlockSpec((_EDGE_BLK, 32), lambda i: (i, 0)),
        out_shape=jax.ShapeDtypeStruct((N_EDGES, 32), jnp.float32),
    )(g, ea, w1ae, w1b, b1b)


def _node_body(x_ref, agg_ref, w2ax_ref, w2aa_ref, b2a_ref, w2b_ref, b2b_ref,
               w1a_ref, b1a_ref, comb_ref, np_ref):
    x = x_ref[...]
    agg = agg_ref[...]
    h = jax.lax.dot_general(x, w2ax_ref[...], (((1,), (0,)), ((), ())),
                            preferred_element_type=jnp.float32)
    h = h + jax.lax.dot_general(agg, w2aa_ref[...], (((1,), (0,)), ((), ())),
                                preferred_element_type=jnp.float32)
    h = jnp.maximum(h + b2a_ref[...], 0.0)
    comb = jax.lax.dot_general(h, w2b_ref[...], (((1,), (0,)), ((), ())),
                               preferred_element_type=jnp.float32)
    comb = jnp.maximum(comb + b2b_ref[...], 0.0)
    comb_ref[...] = comb
    x_next = jnp.concatenate([x[:, :11], comb], axis=1)
    npre = jax.lax.dot_general(x_next, w1a_ref[...], (((1,), (0,)), ((), ())),
                               preferred_element_type=jnp.float32)
    np_ref[...] = npre + b1a_ref[...]


def _node_mlp(x, agg, w2ax, w2aa, b2a, w2b, b2b, w1a, b1a):
    nblk = N_NODES // _NODE_BLK
    return pl.pallas_call(
        _node_body,
        grid=(nblk,),
        in_specs=[
            pl.BlockSpec((_NODE_BLK, 12), lambda i: (i, 0)),
            pl.BlockSpec((_NODE_BLK, 32), lambda i: (i, 0)),
            pl.BlockSpec((12, 16), lambda i: (0, 0)),
            pl.BlockSpec((32, 16), lambda i: (0, 0)),
            pl.BlockSpec((1, 16), lambda i: (0, 0)),
            pl.BlockSpec((16, 1), lambda i: (0, 0)),
            pl.BlockSpec((1, 1), lambda i: (0, 0)),
            pl.BlockSpec((12, 16), lambda i: (0, 0)),
            pl.BlockSpec((1, 16), lambda i: (0, 0)),
        ],
        out_specs=[
            pl.BlockSpec((_NODE_BLK, 1), lambda i: (i, 0)),
            pl.BlockSpec((_NODE_BLK, 16), lambda i: (i, 0)),
        ],
        out_shape=[
            jax.ShapeDtypeStruct((N_NODES, 1), jnp.float32),
            jax.ShapeDtypeStruct((N_NODES, 16), jnp.float32),
        ],
    )(x, agg, w2ax, w2aa, b2a, w2b, b2b, w1a, b1a)


def _pre_body(x_ref, w1a_ref, b1a_ref, np_ref):
    npre = jax.lax.dot_general(x_ref[...], w1a_ref[...], (((1,), (0,)), ((), ())),
                               preferred_element_type=jnp.float32)
    np_ref[...] = npre + b1a_ref[...]


def _node_pre(x, w1a, b1a):
    nblk = N_NODES // _NODE_BLK
    return pl.pallas_call(
        _pre_body,
        grid=(nblk,),
        in_specs=[
            pl.BlockSpec((_NODE_BLK, 12), lambda i: (i, 0)),
            pl.BlockSpec((12, 16), lambda i: (0, 0)),
            pl.BlockSpec((1, 16), lambda i: (0, 0)),
        ],
        out_specs=pl.BlockSpec((_NODE_BLK, 16), lambda i: (i, 0)),
        out_shape=jax.ShapeDtypeStruct((N_NODES, 16), jnp.float32),
    )(x, w1a, b1a)


def kernel(x, edge_index, edge_attr, W1a, b1a, W1b, b1b, W2a, b2a, W2b, b2b):
    src = edge_index[0].astype(jnp.int32)
    dst = edge_index[1].astype(jnp.int32)
    ea = edge_attr

    w1a_x = W1a[:12]
    w1a_e = W1a[12:14]
    b1a_r = b1a.reshape(1, 16)
    b1b_r = b1b.reshape(1, 32)
    w2a_x = W2a[:12]
    w2a_a = W2a[12:44]
    b2a_r = b2a.reshape(1, 16)
    b2b_r = b2b.reshape(1, 1)

    xc = x[:, :11]
    npre = _node_pre(x, w1a_x, b1a_r)
    x_cur = x
    for _ in range(3):
        g = _sc_gather(jnp.reshape(npre, (12500, 128)), src)
        msg = _edge_mlp(g, ea, w1a_e, W1b, b1b_r)
        agg = jax.ops.segment_max(msg, dst, num_segments=N_NODES)
        agg = jnp.maximum(agg, 0.0)
        comb, npre = _node_mlp(x_cur, agg, w2a_x, w2a_a, b2a_r, W2b,
                               b2b_r, w1a_x, b1a_r)
        x_cur = jnp.concatenate([xc, comb], axis=1)
    return x_cur
